# baseline (speedup 1.0000x reference)
import jax
import jax.numpy as jnp
from jax import lax
from jax.experimental import pallas as pl
from jax.experimental.pallas import tpu as pltpu
from jax.experimental.pallas import tpu_sc as plsc

_EMB = 64
_CLASSES = 1000
_B = 1024
_TOK = 1000
_CHUNK = 40
_CPB = _TOK // _CHUNK
_NW = 32
_BPW = _B // _NW
_CPW = _BPW * _CPB


def _emb_body(table_hbm, idx_hbm, out_hbm, idx_v, rows_v, stage_v, sem):
    c = lax.axis_index("c")
    s = lax.axis_index("s")
    wid = s * 2 + c
    pltpu.sync_copy(idx_hbm.at[pl.ds(wid * _CPW, _CPW)], idx_v)

    def batch_body(i, carry):
        def chunk_body(k, accs):
            pltpu.async_copy(table_hbm.at[idx_v.at[i * _CPB + k]], rows_v, sem).wait()
            accs = list(accs)
            for r in range(_CHUNK):
                p = (r % 2) * 4
                for cb in range(4):
                    accs[p + cb] = accs[p + cb] + rows_v[r, pl.ds(16 * cb, 16)]
            return tuple(accs)

        zero = jnp.zeros((16,), jnp.float32)
        accs = lax.fori_loop(0, _CPB, chunk_body, (zero,) * 8)
        for cb in range(4):
            stage_v[i, pl.ds(16 * cb, 16)] = accs[cb] + accs[4 + cb]
        return carry

    lax.fori_loop(0, _BPW, batch_body, 0)
    pltpu.sync_copy(stage_v, out_hbm.at[pl.ds(wid * _BPW, _BPW)])


def _embedding_sums(table, idx):
    k = pl.kernel(
        _emb_body,
        out_type=jax.ShapeDtypeStruct((_B, _EMB), jnp.float32),
        mesh=plsc.VectorSubcoreMesh(core_axis_name="c", subcore_axis_name="s"),
        compiler_params=pltpu.CompilerParams(use_tc_tiling_on_sc=False),
        scratch_types=[
            pltpu.VMEM((_CPW, _CHUNK), jnp.int32),
            pltpu.VMEM((_CHUNK, _EMB), jnp.float32),
            pltpu.VMEM((_BPW, _EMB), jnp.float32),
            pltpu.SemaphoreType.DMA,
        ],
    )
    return k(table, idx)


def _linear_body(x_ref, w_ref, b_ref, o_ref):
    x = x_ref[...] * (1.0 / _TOK)
    o_ref[...] = (
        jnp.dot(x, w_ref[...], preferred_element_type=jnp.float32) + b_ref[...]
    )


def _linear(sums, wt, b2):
    return pl.pallas_call(
        _linear_body,
        out_shape=jax.ShapeDtypeStruct((_B, _CLASSES), jnp.float32),
    )(sums, wt, b2)


def kernel(tok_lists_batch, table, W, b):
    idx = tok_lists_batch.reshape(_B * _CPB, _CHUNK).astype(jnp.int32)
    sums = _embedding_sums(table, idx)
    return _linear(sums, W.T, b.reshape(1, _CLASSES))


# speedup vs baseline: 1.3159x; 1.3159x over previous
import jax
import jax.numpy as jnp
from jax import lax
from jax.experimental import pallas as pl
from jax.experimental.pallas import tpu as pltpu
from jax.experimental.pallas import tpu_sc as plsc

_EMB = 64
_CLASSES = 1000
_B = 1024
_TOK = 1000
_CHUNK = 40
_CPB = _TOK // _CHUNK
_NW = 32
_BPW = _B // _NW
_CPW = _BPW * _CPB


def _emb_body(table_hbm, idx_hbm, out_hbm, idx_v, rows_a, rows_b, stage_v,
              sem_a, sem_b):
    c = lax.axis_index("c")
    s = lax.axis_index("s")
    wid = s * 2 + c
    pltpu.sync_copy(idx_hbm.at[pl.ds(wid * _CPW, _CPW)], idx_v)

    def start(g, buf, sem):
        pltpu.async_copy(table_hbm.at[idx_v.at[g]], buf, sem)

    def wait(buf, sem):
        pltpu.make_async_copy(table_hbm.at[idx_v.at[0]], buf, sem).wait()

    def accumulate(buf, accs):
        accs = list(accs)
        for r in range(_CHUNK):
            p = (r % 2) * 4
            for cb in range(4):
                accs[p + cb] = accs[p + cb] + buf[r, pl.ds(16 * cb, 16)]
        return accs

    def boundary(g, accs):
        bnd = lax.rem(g, _CPB) == _CPB - 1
        i = lax.div(g, _CPB)

        @pl.when(bnd)
        def _():
            for cb in range(4):
                stage_v[i, pl.ds(16 * cb, 16)] = accs[cb] + accs[4 + cb]

        zero = jnp.zeros((16,), jnp.float32)
        return tuple(jnp.where(bnd, zero, a) for a in accs)

    start(0, rows_a, sem_a)

    def pair_body(j, accs):
        ga = 2 * j
        gb = 2 * j + 1
        start(gb, rows_b, sem_b)
        wait(rows_a, sem_a)
        accs = accumulate(rows_a, accs)
        accs = boundary(ga, accs)

        @pl.when(j < _CPW // 2 - 1)
        def _():
            start(ga + 2, rows_a, sem_a)

        wait(rows_b, sem_b)
        accs = accumulate(rows_b, accs)
        accs = boundary(gb, accs)
        return accs

    zero = jnp.zeros((16,), jnp.float32)
    lax.fori_loop(0, _CPW // 2, pair_body, (zero,) * 8)
    pltpu.sync_copy(stage_v, out_hbm.at[pl.ds(wid * _BPW, _BPW)])


def _embedding_sums(table, idx):
    k = pl.kernel(
        _emb_body,
        out_type=jax.ShapeDtypeStruct((_B, _EMB), jnp.float32),
        mesh=plsc.VectorSubcoreMesh(core_axis_name="c", subcore_axis_name="s"),
        compiler_params=pltpu.CompilerParams(use_tc_tiling_on_sc=False),
        scratch_types=[
            pltpu.VMEM((_CPW, _CHUNK), jnp.int32),
            pltpu.VMEM((_CHUNK, _EMB), jnp.float32),
            pltpu.VMEM((_CHUNK, _EMB), jnp.float32),
            pltpu.VMEM((_BPW, _EMB), jnp.float32),
            pltpu.SemaphoreType.DMA,
            pltpu.SemaphoreType.DMA,
        ],
    )
    return k(table, idx)


def _linear_body(x_ref, w_ref, b_ref, o_ref):
    x = x_ref[...] * (1.0 / _TOK)
    o_ref[...] = (
        jnp.dot(x, w_ref[...], preferred_element_type=jnp.float32) + b_ref[...]
    )


def _linear(sums, wt, b2):
    return pl.pallas_call(
        _linear_body,
        out_shape=jax.ShapeDtypeStruct((_B, _CLASSES), jnp.float32),
    )(sums, wt, b2)


def kernel(tok_lists_batch, table, W, b):
    idx = tok_lists_batch.reshape(_B * _CPB, _CHUNK).astype(jnp.int32)
    sums = _embedding_sums(table, idx)
    return _linear(sums, W.T, b.reshape(1, _CLASSES))


# speedup vs baseline: 1.5618x; 1.1869x over previous
import jax
import jax.numpy as jnp
from jax import lax
from jax.experimental import pallas as pl
from jax.experimental.pallas import tpu as pltpu
from jax.experimental.pallas import tpu_sc as plsc

_EMB = 64
_CLASSES = 1000
_B = 1024
_S = 50
_L = 20
_CPB = _S
_NW = 32
_BPW = _B // _NW
_CPW = _BPW * _CPB
_DEPTH = 8


def _emb_body(table_hbm, toks_hbm, out_hbm, idx_v, rows, stage_v, sems):
    c = lax.axis_index("c")
    s = lax.axis_index("s")
    wid = s * 2 + c
    pltpu.sync_copy(toks_hbm.at[pl.ds(wid * _BPW, _BPW)], idx_v)

    def start(g, q):
        i = lax.div(g, _CPB)
        k = lax.rem(g, _CPB)
        pltpu.async_copy(table_hbm.at[idx_v.at[i, k]], rows[q], sems[q])

    def wait(q):
        pltpu.make_async_copy(table_hbm.at[idx_v.at[0, 0]], rows[q], sems[q]).wait()

    def accumulate(q, accs):
        accs = list(accs)
        for r in range(_L):
            p = (r % 2) * 4
            for cb in range(4):
                accs[p + cb] = accs[p + cb] + rows[q][r, pl.ds(16 * cb, 16)]
        return accs

    def boundary(g, accs):
        bnd = lax.rem(g, _CPB) == _CPB - 1
        i = lax.div(g, _CPB)

        @pl.when(bnd)
        def _():
            for cb in range(4):
                stage_v[i, pl.ds(16 * cb, 16)] = accs[cb] + accs[4 + cb]

        zero = jnp.zeros((16,), jnp.float32)
        return tuple(jnp.where(bnd, zero, a) for a in accs)

    for q in range(_DEPTH):
        start(q, q)

    def round_body(j, accs):
        for q in range(_DEPTH):
            g = _DEPTH * j + q
            wait(q)
            accs = accumulate(q, accs)
            accs = boundary(g, accs)

            @pl.when(g + _DEPTH < _CPW)
            def _():
                start(g + _DEPTH, q)

        return tuple(accs)

    zero = jnp.zeros((16,), jnp.float32)
    lax.fori_loop(0, _CPW // _DEPTH, round_body, (zero,) * 8)
    pltpu.sync_copy(stage_v, out_hbm.at[pl.ds(wid * _BPW, _BPW)])


def _embedding_sums(table, toks):
    k = pl.kernel(
        _emb_body,
        out_type=jax.ShapeDtypeStruct((_B, _EMB), jnp.float32),
        mesh=plsc.VectorSubcoreMesh(core_axis_name="c", subcore_axis_name="s"),
        compiler_params=pltpu.CompilerParams(use_tc_tiling_on_sc=False),
        scratch_types=[
            pltpu.VMEM((_BPW, _S, _L), jnp.int32),
            [pltpu.VMEM((_L, _EMB), jnp.float32)] * _DEPTH,
            pltpu.VMEM((_BPW, _EMB), jnp.float32),
            [pltpu.SemaphoreType.DMA] * _DEPTH,
        ],
    )
    return k(table, toks)


def _linear_body(x_ref, w_ref, b_ref, o_ref):
    x = x_ref[...] * (1.0 / (_S * _L))
    o_ref[...] = (
        jnp.dot(x, w_ref[...], preferred_element_type=jnp.float32) + b_ref[...]
    )


def _linear(sums, wt, b2):
    return pl.pallas_call(
        _linear_body,
        out_shape=jax.ShapeDtypeStruct((_B, _CLASSES), jnp.float32),
    )(sums, wt, b2)


def kernel(tok_lists_batch, table, W, b):
    toks = tok_lists_batch.astype(jnp.int32)
    sums = _embedding_sums(table, toks)
    return _linear(sums, W.T, b.reshape(1, _CLASSES))


# speedup vs baseline: 1.6045x; 1.0273x over previous
import jax
import jax.numpy as jnp
from jax import lax
from jax.experimental import pallas as pl
from jax.experimental.pallas import tpu as pltpu
from jax.experimental.pallas import tpu_sc as plsc

_EMB = 64
_CLASSES = 1000
_B = 1024
_S = 50
_L = 20
_CPB = _S
_NW = 32
_BPW = _B // _NW
_LP = 32
_TPW = _BPW * _S * _LP
_CPW = _BPW * _CPB
_DEPTH = 8


def _flatten_body(toks_hbm, out_hbm, sent, flat_v, sems):
    c = lax.axis_index("c")
    s = lax.axis_index("s")
    wid = s * 2 + c

    def start(b, q):
        pltpu.async_copy(toks_hbm.at[wid * _BPW + b], sent[q], sems[q])

    def wait(q):
        pltpu.make_async_copy(toks_hbm.at[0], sent[q], sems[q]).wait()

    def flatten(b, q):
        for k in range(_S):
            base = (b * _S + k) * _LP
            flat_v[pl.ds(base, 16)] = sent[q][k, pl.ds(0, 16)]
            flat_v[pl.ds(base + 16, 16)] = lax.rev(
                sent[q][k, pl.ds(4, 16)], (0,)
            )

    start(0, 0)

    def pair_body(j, carry):
        ba = 2 * j
        start(ba + 1, 1)
        wait(0)
        flatten(ba, 0)

        @pl.when(ba + 2 < _BPW)
        def _():
            start(ba + 2, 0)

        wait(1)
        flatten(ba + 1, 1)
        return carry

    lax.fori_loop(0, _BPW // 2, pair_body, 0)
    pltpu.sync_copy(flat_v, out_hbm.at[pl.ds(wid * _TPW, _TPW)])


def _flatten_toks(toks):
    k = pl.kernel(
        _flatten_body,
        out_type=jax.ShapeDtypeStruct((_B * _S * _LP,), jnp.int32),
        mesh=plsc.VectorSubcoreMesh(core_axis_name="c", subcore_axis_name="s"),
        scratch_types=[
            [pltpu.VMEM((_S, _L), jnp.int32)] * 2,
            pltpu.VMEM((_TPW,), jnp.int32),
            [pltpu.SemaphoreType.DMA] * 2,
        ],
    )
    return k(toks)


def _emb_body(table_hbm, idx_hbm, out_hbm, idx_v, rows, stage_v, sems):
    c = lax.axis_index("c")
    s = lax.axis_index("s")
    wid = s * 2 + c
    pltpu.sync_copy(idx_hbm.at[pl.ds(wid * _TPW, _TPW)], idx_v)

    def start(g, q):
        pltpu.async_copy(
            table_hbm.at[idx_v.at[pl.ds(g * _LP, _L)]], rows[q], sems[q]
        )

    def wait(q):
        pltpu.make_async_copy(
            table_hbm.at[idx_v.at[pl.ds(0, _L)]], rows[q], sems[q]
        ).wait()

    def accumulate(q, accs):
        accs = list(accs)
        for r in range(_L):
            p = (r % 2) * 4
            for cb in range(4):
                accs[p + cb] = accs[p + cb] + rows[q][r, pl.ds(16 * cb, 16)]
        return accs

    def boundary(g, accs):
        bnd = lax.rem(g, _CPB) == _CPB - 1
        i = lax.div(g, _CPB)

        @pl.when(bnd)
        def _():
            for cb in range(4):
                stage_v[i, pl.ds(16 * cb, 16)] = accs[cb] + accs[4 + cb]

        zero = jnp.zeros((16,), jnp.float32)
        return tuple(jnp.where(bnd, zero, a) for a in accs)

    for q in range(_DEPTH):
        start(q, q)

    def round_body(j, accs):
        for q in range(_DEPTH):
            g = _DEPTH * j + q
            wait(q)
            accs = accumulate(q, accs)
            accs = boundary(g, accs)

            @pl.when(g + _DEPTH < _CPW)
            def _():
                start(g + _DEPTH, q)

        return tuple(accs)

    zero = jnp.zeros((16,), jnp.float32)
    lax.fori_loop(0, _CPW // _DEPTH, round_body, (zero,) * 8)
    pltpu.sync_copy(stage_v, out_hbm.at[pl.ds(wid * _BPW, _BPW)])


def _embedding_sums(table, idx):
    k = pl.kernel(
        _emb_body,
        out_type=jax.ShapeDtypeStruct((_B, _EMB), jnp.float32),
        mesh=plsc.VectorSubcoreMesh(core_axis_name="c", subcore_axis_name="s"),
        compiler_params=pltpu.CompilerParams(use_tc_tiling_on_sc=False),
        scratch_types=[
            pltpu.VMEM((_TPW,), jnp.int32),
            [pltpu.VMEM((_L, _EMB), jnp.float32)] * _DEPTH,
            pltpu.VMEM((_BPW, _EMB), jnp.float32),
            [pltpu.SemaphoreType.DMA] * _DEPTH,
        ],
    )
    return k(table, idx)


def _linear_body(x_ref, w_ref, b_ref, o_ref):
    x = x_ref[...] * (1.0 / (_S * _L))
    o_ref[...] = (
        jnp.dot(x, w_ref[...], preferred_element_type=jnp.float32) + b_ref[...]
    )


def _linear(sums, wt, b2):
    return pl.pallas_call(
        _linear_body,
        out_shape=jax.ShapeDtypeStruct((_B, _CLASSES), jnp.float32),
    )(sums, wt, b2)


def kernel(tok_lists_batch, table, W, b):
    toks = tok_lists_batch.astype(jnp.int32)
    idx = _flatten_toks(toks)
    sums = _embedding_sums(table, idx)
    return _linear(sums, W.T, b.reshape(1, _CLASSES))
